# baseline (speedup 1.0000x reference)
import functools

import jax
import jax.numpy as jnp
import numpy as np
from jax import lax
from jax.experimental import pallas as pl
from jax.experimental.pallas import tpu as pltpu
from jax.experimental.pallas import tpu_sc as plsc

LOC_DELTA = np.float32(0.05)
MAX_INDEX = 512.0
MAX_VALUE_Z = np.float32(10.0)
MAX_LOC = np.float32((MAX_INDEX - 1.0) * 0.05)
MAGIC = np.float32(8388608.0)

B, K = 16384, 50
NC, NS, L = 2, 16, 16
NW = NC * NS
ROWS_PER_W = B // NW
CHUNK_ROWS = 64
N_CHUNKS = ROWS_PER_W // CHUNK_ROWS
BLK_ROWS = 8
N_BLK = CHUNK_ROWS // BLK_ROWS
STEPS = BLK_ROWS * K // L

IN_W = CHUNK_ROWS * K * 3
XY_W = CHUNK_ROWS * K * 2
Z_W = CHUNK_ROWS * K
BLK_IN = BLK_ROWS * K * 3
BLK_XY = BLK_ROWS * K * 2
BLK_Z = BLK_ROWS * K

_ROWPAT = (np.arange(BLK_Z, dtype=np.int32) // K).astype(np.int32)


def _sc_body(in_hbm, rp_hbm, xy_hbm, z_hbm, idx_hbm, in_v, xy_v, z_v, idx_v, rp_v):
    wid = lax.axis_index("s") * NC + lax.axis_index("c")
    iota = lax.iota(jnp.int32, L)
    v3 = iota * 3
    v3a = v3 + 1
    v3b = v3 + 2
    v2 = iota * 2
    v2a = v2 + 1
    pltpu.sync_copy(rp_hbm, rp_v)

    def chunk_body(c, carry):
        base_row = wid * ROWS_PER_W + c * CHUNK_ROWS
        pltpu.sync_copy(in_hbm.at[pl.ds(base_row * (K * 3), IN_W)], in_v)

        def blk_body(bl, carry2):
            in_off = bl * BLK_IN
            xy_off = bl * BLK_XY
            z_off = bl * BLK_Z
            brow0 = base_row + bl * BLK_ROWS
            for j in range(STEPS):
                o = in_off + 48 * j
                gix = v3 + o
                giy = v3a + o
                giz = v3b + o
                gx = plsc.load_gather(in_v, [gix])
                gy = plsc.load_gather(in_v, [giy])
                gz = plsc.load_gather(in_v, [giz])
                x = jnp.minimum(gx, MAX_LOC)
                y = jnp.minimum(gy, MAX_LOC)
                rx = (x / LOC_DELTA + MAGIC) - MAGIC
                ry = (y / LOC_DELTA + MAGIC) - MAGIC
                rx = jnp.minimum(jnp.maximum(rx, np.float32(0.0)), np.float32(511.0))
                ry = jnp.minimum(jnp.maximum(ry, np.float32(0.0)), np.float32(511.0))
                ixq = rx.astype(jnp.int32)
                iyq = ry.astype(jnp.int32)
                zc = jnp.minimum(gz, MAX_VALUE_Z)
                xo = xy_off + 32 * j
                plsc.store_scatter(xy_v, [v2 + xo], x)
                plsc.store_scatter(xy_v, [v2a + xo], y)
                z_v[pl.ds(z_off + 16 * j, L)] = zc
                bvec = rp_v[pl.ds(16 * j, L)] + brow0
                plsc.store_scatter(idx_v, [gix], bvec)
                plsc.store_scatter(idx_v, [giy], ixq)
                plsc.store_scatter(idx_v, [giz], iyq)
            return carry2

        lax.fori_loop(0, N_BLK, blk_body, 0)
        pltpu.sync_copy(xy_v, xy_hbm.at[pl.ds(base_row * (K * 2), XY_W)])
        pltpu.sync_copy(z_v, z_hbm.at[pl.ds(base_row * K, Z_W)])
        pltpu.sync_copy(idx_v, idx_hbm.at[pl.ds(base_row * (K * 3), IN_W)])
        return carry

    lax.fori_loop(0, N_CHUNKS, chunk_body, 0)


_sc_call = functools.partial(
    pl.kernel,
    mesh=plsc.VectorSubcoreMesh(core_axis_name="c", subcore_axis_name="s"),
    compiler_params=pltpu.CompilerParams(needs_layout_passes=False),
    out_type=[
        jax.ShapeDtypeStruct((B * K * 2,), jnp.float32),
        jax.ShapeDtypeStruct((B * K,), jnp.float32),
        jax.ShapeDtypeStruct((B * K * 3,), jnp.int32),
    ],
    scratch_types=[
        pltpu.VMEM((IN_W,), jnp.float32),
        pltpu.VMEM((XY_W,), jnp.float32),
        pltpu.VMEM((Z_W,), jnp.float32),
        pltpu.VMEM((IN_W,), jnp.int32),
        pltpu.VMEM((BLK_Z,), jnp.int32),
    ],
)(_sc_body)


def kernel(inputs):
    flat = inputs.reshape(B * K * 3)
    xy, z, idx = _sc_call(flat, jnp.asarray(_ROWPAT))
    return (xy.reshape(B, K, 2), z, idx.reshape(B * K, 3))


# speedup vs baseline: 26.6448x; 26.6448x over previous
import functools

import jax
import jax.numpy as jnp
import numpy as np
from jax import lax
from jax.experimental import pallas as pl
from jax.experimental.pallas import tpu as pltpu
from jax.experimental.pallas import tpu_sc as plsc

LOC_DELTA = np.float32(0.05)
MAX_INDEX = 512.0
MAX_VALUE_Z = np.float32(10.0)
MAX_LOC = np.float32((MAX_INDEX - 1.0) * 0.05)
MAGIC = np.float32(8388608.0)

B, K = 16384, 50
NC, NS, L = 2, 16, 16
NW = NC * NS
BPW = B // NW
CB = 128
NCH = BPW // CB
CKP = CB * K
NGRP = CKP // 128

_P = np.arange(CKP, dtype=np.int32)
_KPAT = (_P % K).astype(np.int32)
_BPAT = (_P // K).astype(np.int32)


def _sc_body(in_hbm, kp_hbm, bp_hbm, xy_hbm, z_hbm, idx_hbm,
             x_v, y_v, z_v, ix_v, iy_v, xy_o, z_o, idx_o, kp_v, bp_v):
    wid = lax.axis_index("s") * NC + lax.axis_index("c")
    pltpu.sync_copy(kp_hbm, kp_v)
    pltpu.sync_copy(bp_hbm, bp_v)

    def chunk_body(c, carry):
        b0 = wid * BPW + c * CB
        g0 = b0 * K // 128
        pltpu.sync_copy(in_hbm.at[0, :, pl.ds(b0, CB)], x_v)
        pltpu.sync_copy(in_hbm.at[1, :, pl.ds(b0, CB)], y_v)
        pltpu.sync_copy(in_hbm.at[2, :, pl.ds(b0, CB)], z_v)

        def quant_body(k, carry2):
            for j in range(CB // L):
                sl = pl.ds(16 * j, L)
                x = jnp.minimum(x_v[k, sl], MAX_LOC)
                y = jnp.minimum(y_v[k, sl], MAX_LOC)
                xy_o[k, 0, 0, sl] = x
                xy_o[k, 0, 1, sl] = y
                rx = (x / LOC_DELTA + MAGIC) - MAGIC
                ry = (y / LOC_DELTA + MAGIC) - MAGIC
                rx = jnp.minimum(jnp.maximum(rx, np.float32(0.0)), np.float32(511.0))
                ry = jnp.minimum(jnp.maximum(ry, np.float32(0.0)), np.float32(511.0))
                ix_v[k, sl] = rx.astype(jnp.int32)
                iy_v[k, sl] = ry.astype(jnp.int32)
            return carry2

        lax.fori_loop(0, K, quant_body, 0)

        def grp_body(g, carry3):
            for j in range(128 // L):
                sl = pl.ds(16 * j, L)
                tsl = pl.ds(g * 128 + 16 * j, L)
                kp = kp_v[tsl]
                bp = bp_v[tsl]
                zg = plsc.load_gather(z_v, [kp, bp])
                z_o[tsl] = jnp.minimum(zg, MAX_VALUE_Z)
                idx_o[g, 0, sl] = bp + b0
                idx_o[g, 1, sl] = plsc.load_gather(ix_v, [kp, bp])
                idx_o[g, 2, sl] = plsc.load_gather(iy_v, [kp, bp])
                idx_o[g, 3, sl] = bp
            return carry3

        lax.fori_loop(0, NGRP, grp_body, 0)

        pltpu.sync_copy(xy_o, xy_hbm.at[:, pl.ds(b0 // 128, 1), :, :])
        pltpu.sync_copy(z_o, z_hbm.at[pl.ds(b0 * K, CKP)])
        pltpu.sync_copy(idx_o, idx_hbm.at[pl.ds(g0, NGRP)])
        return carry

    lax.fori_loop(0, NCH, chunk_body, 0)


_sc_call = functools.partial(
    pl.kernel,
    mesh=plsc.VectorSubcoreMesh(core_axis_name="c", subcore_axis_name="s"),
    compiler_params=pltpu.CompilerParams(
        needs_layout_passes=False, use_tc_tiling_on_sc=False
    ),
    out_type=[
        jax.ShapeDtypeStruct((K, 128, 2, 128), jnp.float32),
        jax.ShapeDtypeStruct((B * K,), jnp.float32),
        jax.ShapeDtypeStruct((B * K // 128, 4, 128), jnp.int32),
    ],
    scratch_types=[
        pltpu.VMEM((K, CB), jnp.float32),
        pltpu.VMEM((K, CB), jnp.float32),
        pltpu.VMEM((K, CB), jnp.float32),
        pltpu.VMEM((K, CB), jnp.int32),
        pltpu.VMEM((K, CB), jnp.int32),
        pltpu.VMEM((K, 1, 2, 128), jnp.float32),
        pltpu.VMEM((CKP,), jnp.float32),
        pltpu.VMEM((NGRP, 4, 128), jnp.int32),
        pltpu.VMEM((CKP,), jnp.int32),
        pltpu.VMEM((CKP,), jnp.int32),
    ],
)(_sc_body)


def kernel(inputs):
    tin = jnp.transpose(inputs, (2, 1, 0))
    xy4, z, idx4 = _sc_call(tin, jnp.asarray(_KPAT), jnp.asarray(_BPAT))
    xy = xy4.transpose(1, 3, 0, 2).reshape(B, K, 2)
    idx = idx4[:, 0:3, :].transpose(0, 2, 1).reshape(B * K, 3)
    return (xy, z, idx)


# speedup vs baseline: 54.3770x; 2.0408x over previous
import functools

import jax
import jax.numpy as jnp
import numpy as np
from jax import lax
from jax.experimental import pallas as pl
from jax.experimental.pallas import tpu as pltpu
from jax.experimental.pallas import tpu_sc as plsc

LOC_DELTA = np.float32(0.05)
MAX_INDEX = 512.0
MAX_VALUE_Z = np.float32(10.0)
MAX_LOC = np.float32((MAX_INDEX - 1.0) * 0.05)
MAGIC = np.float32(8388608.0)

B, K = 16384, 50
NC, NS, L = 2, 16, 16
NW = NC * NS
BPW = B // NW
CB = 128
NCH = BPW // CB
CKP = CB * K

_I = np.arange(CKP, dtype=np.int32)
_KP = (_I % CB) * K + _I // CB
_T1 = _KP.astype(np.int32)
_T2 = ((_KP // 128) * 512 + 128 + _KP % 128).astype(np.int32)


def _sc_body(in_hbm, t1_hbm, t2_hbm, xy_hbm, z_hbm, idx_hbm,
             x0, y0, z0, x1, y1, z1, zo0, zo1, io0, io1, t1_v, t2_v,
             sin0, sin1, sxy0, sxy1, szo0, szo1, sio0, sio1):
    wid = lax.axis_index("s") * NC + lax.axis_index("c")
    iota = lax.iota(jnp.int32, L)
    pltpu.sync_copy(t1_hbm, t1_v)
    pltpu.sync_copy(t2_hbm, t2_v)

    bufs = [(x0, y0, z0), (x1, y1, z1)]
    zos, ios = [zo0, zo1], [io0, io1]
    sins, sxys, szos, sios = [sin0, sin1], [sxy0, sxy1], [szo0, szo1], [sio0, sio1]

    def start_in(c):
        s = c % 2
        b0 = wid * BPW + c * CB
        return [
            pltpu.async_copy(in_hbm.at[p, :, pl.ds(b0, CB)], bufs[s][p], sins[s])
            for p in range(3)
        ]

    def compute(c):
        s = c % 2
        xv, yv, zv = bufs[s]
        zo, io = zos[s], ios[s]
        b0 = wid * BPW + c * CB

        def kbody(k, carry):
            for j in range(CB // L):
                sl = pl.ds(16 * j, L)
                tsl = pl.ds(k * CB + 16 * j, L)
                x = jnp.minimum(xv[k, sl], MAX_LOC)
                y = jnp.minimum(yv[k, sl], MAX_LOC)
                xv[k, sl] = x
                yv[k, sl] = y
                rx = (x / LOC_DELTA + MAGIC) - MAGIC
                ry = (y / LOC_DELTA + MAGIC) - MAGIC
                rx = jnp.minimum(jnp.maximum(rx, np.float32(0.0)), np.float32(511.0))
                ry = jnp.minimum(jnp.maximum(ry, np.float32(0.0)), np.float32(511.0))
                zc = jnp.minimum(zv[k, sl], MAX_VALUE_Z)
                t1 = t1_v[tsl]
                t2 = t2_v[tsl]
                plsc.store_scatter(zo, [t1], zc)
                plsc.store_scatter(io, [t2 - 128], iota + (b0 + 16 * j))
                plsc.store_scatter(io, [t2], rx.astype(jnp.int32))
                plsc.store_scatter(io, [t2 + 128], ry.astype(jnp.int32))
            return carry

        lax.fori_loop(0, K, kbody, 0)

    in_h = {0: start_in(0)}
    xy_h, zo_h, io_h = {}, {}, {}
    for c in range(NCH):
        s = c % 2
        b0 = wid * BPW + c * CB
        for h in in_h.pop(c):
            h.wait()
        if c + 1 < NCH:
            if c - 1 in xy_h:
                for h in xy_h.pop(c - 1):
                    h.wait()
            in_h[c + 1] = start_in(c + 1)
        if c - 2 in zo_h:
            zo_h.pop(c - 2).wait()
            io_h.pop(c - 2).wait()
        compute(c)
        xv, yv, _ = bufs[s]
        g = b0 // 128
        xy_h[c] = [
            pltpu.async_copy(xv, xy_hbm.at[:, g, 0, :], sxys[s]),
            pltpu.async_copy(yv, xy_hbm.at[:, g, 1, :], sxys[s]),
        ]
        zo_h[c] = pltpu.async_copy(zos[s], z_hbm.at[pl.ds(b0 * K, CKP)], szos[s])
        io_h[c] = pltpu.async_copy(ios[s], idx_hbm.at[pl.ds(b0 * K * 4, CKP * 4)], sios[s])
    for hs in xy_h.values():
        for h in hs:
            h.wait()
    for h in zo_h.values():
        h.wait()
    for h in io_h.values():
        h.wait()


_sc_call = functools.partial(
    pl.kernel,
    mesh=plsc.VectorSubcoreMesh(core_axis_name="c", subcore_axis_name="s"),
    compiler_params=pltpu.CompilerParams(
        needs_layout_passes=False, use_tc_tiling_on_sc=False
    ),
    out_type=[
        jax.ShapeDtypeStruct((K, 128, 2, 128), jnp.float32),
        jax.ShapeDtypeStruct((B * K,), jnp.float32),
        jax.ShapeDtypeStruct((B * K * 4,), jnp.int32),
    ],
    scratch_types=[
        pltpu.VMEM((K, CB), jnp.float32),
        pltpu.VMEM((K, CB), jnp.float32),
        pltpu.VMEM((K, CB), jnp.float32),
        pltpu.VMEM((K, CB), jnp.float32),
        pltpu.VMEM((K, CB), jnp.float32),
        pltpu.VMEM((K, CB), jnp.float32),
        pltpu.VMEM((CKP,), jnp.float32),
        pltpu.VMEM((CKP,), jnp.float32),
        pltpu.VMEM((CKP * 4,), jnp.int32),
        pltpu.VMEM((CKP * 4,), jnp.int32),
        pltpu.VMEM((CKP,), jnp.int32),
        pltpu.VMEM((CKP,), jnp.int32),
        pltpu.SemaphoreType.DMA,
        pltpu.SemaphoreType.DMA,
        pltpu.SemaphoreType.DMA,
        pltpu.SemaphoreType.DMA,
        pltpu.SemaphoreType.DMA,
        pltpu.SemaphoreType.DMA,
        pltpu.SemaphoreType.DMA,
        pltpu.SemaphoreType.DMA,
    ],
)(_sc_body)


def kernel(inputs):
    tin = jnp.transpose(inputs, (2, 1, 0))
    xy4, z, idxf = _sc_call(tin, jnp.asarray(_T1), jnp.asarray(_T2))
    xy = xy4.transpose(1, 3, 0, 2).reshape(B, K, 2)
    idx = idxf.reshape(B * K // 128, 4, 128)[:, 0:3, :].transpose(0, 2, 1).reshape(B * K, 3)
    return (xy, z, idx)


# speedup vs baseline: 64.6195x; 1.1884x over previous
import functools

import jax
import jax.numpy as jnp
import numpy as np
from jax import lax
from jax.experimental import pallas as pl
from jax.experimental.pallas import tpu as pltpu
from jax.experimental.pallas import tpu_sc as plsc

LOC_DELTA = np.float32(0.05)
MAX_INDEX = 512.0
MAX_VALUE_Z = np.float32(10.0)
MAX_LOC = np.float32((MAX_INDEX - 1.0) * 0.05)
MAGIC = np.float32(8388608.0)

B, K = 16384, 50
NC, NS, L = 2, 16, 16
NW = NC * NS
BPW = B // NW
CB = 128
NCH = BPW // CB
CKP = CB * K

_I = np.arange(CKP, dtype=np.int32)
_KP = (_I % CB) * K + _I // CB
_T2 = ((_KP // 128) * 512 + 128 + _KP % 128).astype(np.int32)


def _sc_body(in_hbm, t2_hbm, xy_hbm, z_hbm, idx_hbm,
             x0, y0, z0, x1, y1, z1, zo0, zo1, io0, io1, t2_v,
             sin0, sin1, sxy0, sxy1, szo0, szo1, sio0, sio1):
    wid = lax.axis_index("s") * NC + lax.axis_index("c")
    iota = lax.iota(jnp.int32, L)
    pltpu.sync_copy(t2_hbm, t2_v)

    bufs = [(x0, y0, z0), (x1, y1, z1)]
    zos, ios = [zo0, zo1], [io0, io1]
    sins, sxys, szos, sios = [sin0, sin1], [sxy0, sxy1], [szo0, szo1], [sio0, sio1]

    def start_in(c):
        s = c % 2
        b0 = wid * BPW + c * CB
        return [
            pltpu.async_copy(in_hbm.at[p, :, pl.ds(b0, CB)], bufs[s][p], sins[s])
            for p in range(3)
        ]

    v50 = lax.iota(jnp.int32, L) * K

    def compute(c):
        s = c % 2
        xv, yv, zv = bufs[s]
        zo, io = zos[s], ios[s]
        b0 = wid * BPW + c * CB

        @plsc.parallel_loop(0, K, 1, unroll=2)
        def kbody(k):
            for j in range(CB // L):
                sl = pl.ds(16 * j, L)
                tsl = pl.ds(k * CB + 16 * j, L)
                x = jnp.minimum(xv[k, sl], MAX_LOC)
                y = jnp.minimum(yv[k, sl], MAX_LOC)
                xv[k, sl] = x
                yv[k, sl] = y
                rx = (x / LOC_DELTA + MAGIC) - MAGIC
                ry = (y / LOC_DELTA + MAGIC) - MAGIC
                rx = jnp.minimum(jnp.maximum(rx, np.float32(0.0)), np.float32(511.0))
                ry = jnp.minimum(jnp.maximum(ry, np.float32(0.0)), np.float32(511.0))
                zc = jnp.minimum(zv[k, sl], MAX_VALUE_Z)
                t1 = v50 + (k + 800 * j)
                t2 = t2_v[tsl]
                plsc.store_scatter(zo, [t1], zc)
                plsc.store_scatter(io, [t2 - 128], iota + (b0 + 16 * j))
                plsc.store_scatter(io, [t2], rx.astype(jnp.int32))
                plsc.store_scatter(io, [t2 + 128], ry.astype(jnp.int32))

    in_h = {0: start_in(0)}
    xy_h, zo_h, io_h = {}, {}, {}
    for c in range(NCH):
        s = c % 2
        b0 = wid * BPW + c * CB
        for h in in_h.pop(c):
            h.wait()
        if c + 1 < NCH:
            if c - 1 in xy_h:
                for h in xy_h.pop(c - 1):
                    h.wait()
            in_h[c + 1] = start_in(c + 1)
        if c - 2 in zo_h:
            zo_h.pop(c - 2).wait()
            io_h.pop(c - 2).wait()
        compute(c)
        xv, yv, _ = bufs[s]
        g = b0 // 128
        xy_h[c] = [
            pltpu.async_copy(xv, xy_hbm.at[:, g, 0, :], sxys[s]),
            pltpu.async_copy(yv, xy_hbm.at[:, g, 1, :], sxys[s]),
        ]
        zo_h[c] = pltpu.async_copy(zos[s], z_hbm.at[pl.ds(b0 * K, CKP)], szos[s])
        io_h[c] = pltpu.async_copy(ios[s], idx_hbm.at[pl.ds(b0 * K * 4, CKP * 4)], sios[s])
    for hs in xy_h.values():
        for h in hs:
            h.wait()
    for h in zo_h.values():
        h.wait()
    for h in io_h.values():
        h.wait()


_sc_call = functools.partial(
    pl.kernel,
    mesh=plsc.VectorSubcoreMesh(core_axis_name="c", subcore_axis_name="s"),
    compiler_params=pltpu.CompilerParams(
        needs_layout_passes=False, use_tc_tiling_on_sc=False
    ),
    out_type=[
        jax.ShapeDtypeStruct((K, 128, 2, 128), jnp.float32),
        jax.ShapeDtypeStruct((B * K,), jnp.float32),
        jax.ShapeDtypeStruct((B * K * 4,), jnp.int32),
    ],
    scratch_types=[
        pltpu.VMEM((K, CB), jnp.float32),
        pltpu.VMEM((K, CB), jnp.float32),
        pltpu.VMEM((K, CB), jnp.float32),
        pltpu.VMEM((K, CB), jnp.float32),
        pltpu.VMEM((K, CB), jnp.float32),
        pltpu.VMEM((K, CB), jnp.float32),
        pltpu.VMEM((CKP,), jnp.float32),
        pltpu.VMEM((CKP,), jnp.float32),
        pltpu.VMEM((CKP * 4,), jnp.int32),
        pltpu.VMEM((CKP * 4,), jnp.int32),
        pltpu.VMEM((CKP,), jnp.int32),
        pltpu.SemaphoreType.DMA,
        pltpu.SemaphoreType.DMA,
        pltpu.SemaphoreType.DMA,
        pltpu.SemaphoreType.DMA,
        pltpu.SemaphoreType.DMA,
        pltpu.SemaphoreType.DMA,
        pltpu.SemaphoreType.DMA,
        pltpu.SemaphoreType.DMA,
    ],
)(_sc_body)


def kernel(inputs):
    tin = jnp.transpose(inputs, (2, 1, 0))
    xy4, z, idxf = _sc_call(tin, jnp.asarray(_T2))
    xy = xy4.transpose(1, 3, 0, 2).reshape(B, K, 2)
    idx = idxf.reshape(B * K // 128, 4, 128)[:, 0:3, :].transpose(0, 2, 1).reshape(B * K, 3)
    return (xy, z, idx)
